# baseline (speedup 1.0000x reference)
import functools

import jax
import jax.numpy as jnp
from jax import lax
from jax.experimental import pallas as pl
from jax.experimental.pallas import tpu as pltpu
from jax.experimental.pallas import tpu_sc as plsc

B = 4096
S = 200
D = 64
N = B * S

NC, NS = 2, 16
NW = NC * NS
ROWS_PER_W = N // NW

SEQ_BLK = 4
R = SEQ_BLK * S
NBLK = ROWS_PER_W // R

_mesh = plsc.VectorSubcoreMesh(
    core_axis_name="c", subcore_axis_name="s", num_cores=NC, num_subcores=NS
)


@functools.partial(
    pl.kernel,
    out_type=jax.ShapeDtypeStruct((N, D), jnp.float32),
    mesh=_mesh,
    compiler_params=pltpu.CompilerParams(use_tc_tiling_on_sc=False),
    scratch_types=[
        pltpu.VMEM((R,), jnp.int32),
        pltpu.VMEM((R, D), jnp.float32),
        pltpu.VMEM((S, D), jnp.float32),
        pltpu.SemaphoreType.DMA,
    ],
)
def _emb_kernel(idx_hbm, table_hbm, pos_hbm, out_hbm, idx_v, rows_v, pos_v, sem):
    wid = lax.axis_index("s") * NC + lax.axis_index("c")
    base = wid * ROWS_PER_W

    pltpu.sync_copy(pos_hbm, pos_v)

    def blk_body(blk, carry):
        b0 = base + blk * R
        pltpu.sync_copy(idx_hbm.at[pl.ds(b0, R)], idx_v)
        pltpu.async_copy(table_hbm.at[idx_v], rows_v, sem).wait()

        def pos_body(p, c2):
            for j in range(D // 16):
                pv = pos_v[p, pl.ds(j * 16, 16)]
                for s in range(SEQ_BLK):
                    r = s * S + p
                    rows_v[r, pl.ds(j * 16, 16)] = (
                        rows_v[r, pl.ds(j * 16, 16)] + pv
                    )
            return c2

        lax.fori_loop(0, S, pos_body, 0)

        pltpu.sync_copy(rows_v, out_hbm.at[pl.ds(b0, R)])
        return carry

    lax.fori_loop(0, NBLK, blk_body, 0)


def kernel(inputs, token_table, pos_table):
    idx = inputs.reshape(N).astype(jnp.int32)
    out = _emb_kernel(idx, token_table, pos_table)
    return out.reshape(B, S, D)
